# baseline (speedup 1.0000x reference)
import functools

import jax
import jax.numpy as jnp
from jax import lax
from jax.experimental import pallas as pl
from jax.experimental.pallas import tpu as pltpu
from jax.experimental.pallas import tpu_sc as plsc

_N = 4096 * 4096
_NW = 32
_PER_W = _N // _NW
_CHUNK = 32768
_NCH = _PER_W // _CHUNK
_NBUF = 2
_UNROLL = 8
_INNER = _CHUNK // (16 * _UNROLL)

_mesh = plsc.VectorSubcoreMesh(core_axis_name="c", subcore_axis_name="s")


@functools.partial(
    pl.kernel,
    mesh=_mesh,
    out_type=jax.ShapeDtypeStruct((_N,), jnp.float32),
    scratch_types=[
        pltpu.VMEM((_CHUNK,), jnp.float32),
        pltpu.VMEM((_CHUNK,), jnp.float32),
        pltpu.VMEM((16,), jnp.float32),
        pltpu.SemaphoreType.DMA,
        pltpu.SemaphoreType.DMA,
    ],
)
def _sc_scale(w_hbm, m_hbm, out_hbm, buf0, buf1, mvec, sem_in, sem_out):
    wid = lax.axis_index("s") * 2 + lax.axis_index("c")
    base = wid * _PER_W
    bufs = (buf0, buf1)

    pltpu.sync_copy(m_hbm, mvec)
    mv = mvec[...]

    def compute(buf):
        def body(j, _):
            b = j * (16 * _UNROLL)
            for u in range(_UNROLL):
                sl = pl.ds(b + u * 16, 16)
                buf[sl] = buf[sl] * mv
            return 0
        lax.fori_loop(0, _INNER, body, 0, unroll=False)

    def start_in(i):
        return pltpu.async_copy(
            w_hbm.at[pl.ds(base + i * _CHUNK, _CHUNK)], bufs[i % _NBUF], sem_in)

    def start_out(i):
        return pltpu.async_copy(
            bufs[i % _NBUF], out_hbm.at[pl.ds(base + i * _CHUNK, _CHUNK)], sem_out)

    in_cp = [None] * _NCH
    out_cp = [None] * _NCH
    in_cp[0] = start_in(0)
    for i in range(_NCH):
        if i + 1 < _NCH:
            if i + 1 >= _NBUF:
                out_cp[i + 1 - _NBUF].wait()
            in_cp[i + 1] = start_in(i + 1)
        in_cp[i].wait()
        compute(bufs[i % _NBUF])
        out_cp[i] = start_out(i)
    for i in range(_NCH - _NBUF, _NCH):
        if i >= 0:
            out_cp[i].wait()


def kernel(weight, mask):
    w_flat = jnp.reshape(weight, (_N,))
    m16 = jnp.broadcast_to(jnp.reshape(mask.astype(jnp.float32), (1,)), (16,))
    out = _sc_scale(w_flat, m16)
    return jnp.reshape(out, (4096, 4096))


# speedup vs baseline: 1.0005x; 1.0005x over previous
import functools

import jax
import jax.numpy as jnp
from jax import lax
from jax.experimental import pallas as pl
from jax.experimental.pallas import tpu as pltpu
from jax.experimental.pallas import tpu_sc as plsc

_N = 4096 * 4096
_NW = 32
_PER_W = _N // _NW
_CHUNK = 32768
_NCH = _PER_W // _CHUNK
_NBUF = 2
_UNROLL = 8
_INNER = _CHUNK // (16 * _UNROLL)

_mesh = plsc.VectorSubcoreMesh(core_axis_name="c", subcore_axis_name="s")


@functools.partial(
    pl.kernel,
    mesh=_mesh,
    out_type=jax.ShapeDtypeStruct((_N,), jnp.float32),
    scratch_types=[
        pltpu.VMEM((_CHUNK,), jnp.float32),
        pltpu.VMEM((_CHUNK,), jnp.float32),
        pltpu.VMEM((16,), jnp.float32),
        pltpu.SemaphoreType.DMA,
        pltpu.SemaphoreType.DMA,
    ],
)
def _sc_scale(w_hbm, m_hbm, out_hbm, buf0, buf1, mvec, sem_in, sem_out):
    wid = lax.axis_index("s") * 2 + lax.axis_index("c")
    base = wid * _PER_W
    bufs = (buf0, buf1)

    pltpu.sync_copy(m_hbm, mvec)
    mv = mvec[...]

    def compute(buf):
        @plsc.parallel_loop(0, _CHUNK, 16, unroll=_UNROLL)
        def _(i):
            sl = pl.ds(i, 16)
            buf[sl] = buf[sl] * mv

    def start_in(i):
        return pltpu.async_copy(
            w_hbm.at[pl.ds(base + i * _CHUNK, _CHUNK)], bufs[i % _NBUF], sem_in)

    def start_out(i):
        return pltpu.async_copy(
            bufs[i % _NBUF], out_hbm.at[pl.ds(base + i * _CHUNK, _CHUNK)], sem_out)

    in_cp = [None] * _NCH
    out_cp = [None] * _NCH
    in_cp[0] = start_in(0)
    for i in range(_NCH):
        if i + 1 < _NCH:
            if i + 1 >= _NBUF:
                out_cp[i + 1 - _NBUF].wait()
            in_cp[i + 1] = start_in(i + 1)
        in_cp[i].wait()
        compute(bufs[i % _NBUF])
        out_cp[i] = start_out(i)
    for i in range(_NCH - _NBUF, _NCH):
        if i >= 0:
            out_cp[i].wait()


def kernel(weight, mask):
    w_flat = jnp.reshape(weight, (_N,))
    m16 = jnp.broadcast_to(jnp.reshape(mask.astype(jnp.float32), (1,)), (16,))
    out = _sc_scale(w_flat, m16)
    return jnp.reshape(out, (4096, 4096))


# speedup vs baseline: 2.5341x; 2.5328x over previous
import functools

import jax
import jax.numpy as jnp
from jax import lax
from jax.experimental import pallas as pl
from jax.experimental.pallas import tpu as pltpu
from jax.experimental.pallas import tpu_sc as plsc

_R, _C = 4096, 4096
_NW = 32
_ROWS_W = _R // _NW
_CHROWS = 8
_NCH = _ROWS_W // _CHROWS
_NBUF = 2
_UNROLL = 8

_mesh = plsc.VectorSubcoreMesh(core_axis_name="c", subcore_axis_name="s")


@functools.partial(
    pl.kernel,
    mesh=_mesh,
    out_type=jax.ShapeDtypeStruct((_R, _C), jnp.float32),
    scratch_types=[
        pltpu.VMEM((_CHROWS, _C), jnp.float32),
        pltpu.VMEM((_CHROWS, _C), jnp.float32),
        pltpu.VMEM((16,), jnp.float32),
        pltpu.SemaphoreType.DMA,
        pltpu.SemaphoreType.DMA,
    ],
)
def _sc_scale(w_hbm, m_hbm, out_hbm, buf0, buf1, mvec, sem_in, sem_out):
    wid = lax.axis_index("s") * 2 + lax.axis_index("c")
    base = wid * _ROWS_W
    bufs = (buf0, buf1)

    pltpu.sync_copy(m_hbm, mvec)
    mv = mvec[...]

    def compute(buf):
        for r in range(_CHROWS):
            @plsc.parallel_loop(0, _C, 16, unroll=_UNROLL)
            def _(c):
                sl = pl.ds(c, 16)
                buf[r, sl] = buf[r, sl] * mv

    def start_in(i):
        return pltpu.async_copy(
            w_hbm.at[pl.ds(base + i * _CHROWS, _CHROWS)], bufs[i % _NBUF], sem_in)

    def start_out(i):
        return pltpu.async_copy(
            bufs[i % _NBUF], out_hbm.at[pl.ds(base + i * _CHROWS, _CHROWS)], sem_out)

    out_cp = [None] * _NCH
    in_cp = [None] * _NCH
    in_cp[0] = start_in(0)
    for i in range(_NCH):
        if i + 1 < _NCH:
            if i + 1 >= _NBUF:
                out_cp[i + 1 - _NBUF].wait()
            in_cp[i + 1] = start_in(i + 1)
        in_cp[i].wait()
        compute(bufs[i % _NBUF])
        out_cp[i] = start_out(i)
    for i in range(_NCH - _NBUF, _NCH):
        out_cp[i].wait()


def kernel(weight, mask):
    m16 = jnp.broadcast_to(jnp.reshape(mask.astype(jnp.float32), (1,)), (16,))
    return _sc_scale(weight, m16)
